# Initial kernel scaffold; baseline (speedup 1.0000x reference)
import jax
import jax.numpy as jnp
from jax.experimental import pallas as pl


def kernel(x, edge_index, batch, W0, b0, g0, be0, W1, b1, g1, be1, W2, b2, g2, be2, gatW, asrc, adst, gatb, sW1, sb1, sW2, sb2, lW1, lb1, lW2, lb2, bpW1, bpb1, bpW2, bpb2, adW1, adb1, adW2, adb2):
    raise NotImplementedError("write your pallas kernel here")


# speedup vs baseline: 11.6475x; 11.6475x over previous
import math

import jax
import jax.numpy as jnp
from jax import lax
from jax.experimental import pallas as pl
from jax.experimental.pallas import tpu as pltpu
from jax.experimental.pallas import tpu_sc as plsc

N = 10000
E = 320000
D = 128
H = 128
HEADS = 4
G = 64

NCORE = 2
NSUB = 16
NW = NCORE * NSUB
EW = E // NW
CH = 80
NCH = EW // CH
CHA = 40
NCHA = EW // CHA
NPAD = 10240
SLAB = NPAD // NSUB
ROWS = 1000
BNC = 1.0 / math.sqrt(1.0 + 1e-5)


def _mesh():
    return plsc.VectorSubcoreMesh(
        core_axis_name="c", subcore_axis_name="s",
        num_cores=NCORE, num_subcores=NSUB)


def _zero_slab(z_hbm, zb, acc, sid):
    for q in range(SLAB // 80):
        pltpu.sync_copy(z_hbm, zb)
        pltpu.sync_copy(zb, acc.at[pl.ds(sid * SLAB + q * 80, 80), :])


def _write_slab(out_hbm, zb, acc, cid, sid):
    for q in range(SLAB // 80):
        r0 = sid * SLAB + q * 80
        pltpu.sync_copy(acc.at[pl.ds(r0, 80), :], zb)
        pltpu.sync_copy(zb, out_hbm.at[cid, pl.ds(r0, 80), :])


def _sc_deg_body(d_hbm, ones_hbm, z_hbm, out_hbm, idxd, ones_v, zb, acc, sem):
    cid = lax.axis_index("c")
    sid = lax.axis_index("s")
    wid = cid * NSUB + sid
    _zero_slab(z_hbm, zb, acc, sid)
    pltpu.sync_copy(ones_hbm, ones_v)
    plsc.subcore_barrier()

    def step(k, carry):
        base = wid * EW + k * CH
        pltpu.sync_copy(d_hbm.at[pl.ds(base, CH)], idxd)
        pltpu.sync_copy(ones_v, acc.at[idxd], add=True)
        return carry

    lax.fori_loop(0, NCH, step, 0)
    plsc.subcore_barrier()
    _write_slab(out_hbm, zb, acc, cid, sid)


def _sc_deg(d_arr, ones128, z128):
    k = pl.kernel(
        _sc_deg_body,
        out_type=jax.ShapeDtypeStruct((NCORE, NPAD, H), jnp.float32),
        mesh=_mesh(),
        scratch_types=[
            pltpu.VMEM((CH,), jnp.int32),
            pltpu.VMEM((CH, H), jnp.float32),
            pltpu.VMEM((80, H), jnp.float32),
            pltpu.VMEM_SHARED((NPAD, H), jnp.float32),
            pltpu.SemaphoreType.DMA,
        ],
    )
    return k(d_arr, ones128, z128)


def _sc_gcn_body(a_hbm, s_hbm, d_hbm, z_hbm, out_hbm,
                 idxs, idxd, rows, zb, acc, sem):
    cid = lax.axis_index("c")
    sid = lax.axis_index("s")
    wid = cid * NSUB + sid
    _zero_slab(z_hbm, zb, acc, sid)
    plsc.subcore_barrier()

    def step(k, carry):
        base = wid * EW + k * CH
        pltpu.sync_copy(s_hbm.at[pl.ds(base, CH)], idxs)
        pltpu.sync_copy(d_hbm.at[pl.ds(base, CH)], idxd)
        pltpu.async_copy(a_hbm.at[idxs], rows, sem).wait()
        pltpu.sync_copy(rows, acc.at[idxd], add=True)
        return carry

    lax.fori_loop(0, NCH, step, 0)
    plsc.subcore_barrier()
    _write_slab(out_hbm, zb, acc, cid, sid)


def _sc_gcn(a, s_arr, d_arr, z128):
    k = pl.kernel(
        _sc_gcn_body,
        out_type=jax.ShapeDtypeStruct((NCORE, NPAD, H), jnp.float32),
        mesh=_mesh(),
        scratch_types=[
            pltpu.VMEM((CH,), jnp.int32),
            pltpu.VMEM((CH,), jnp.int32),
            pltpu.VMEM((CH, H), jnp.float32),
            pltpu.VMEM((80, H), jnp.float32),
            pltpu.VMEM_SHARED((NPAD, H), jnp.float32),
            pltpu.SemaphoreType.DMA,
        ],
    )
    return k(a, s_arr, d_arr, z128)


def _sc_att_body(es_hbm, ed_hbm, s_hbm, d_hbm, z_hbm, ee_hbm, den_hbm,
                 idxs, idxd, esv, edv, eev, zb, acc, sem):
    cid = lax.axis_index("c")
    sid = lax.axis_index("s")
    wid = cid * NSUB + sid
    _zero_slab(z_hbm, zb, acc, sid)
    plsc.subcore_barrier()

    def step(k, carry):
        base = wid * EW + k * CHA
        pltpu.sync_copy(s_hbm.at[pl.ds(base, CHA)], idxs)
        pltpu.sync_copy(d_hbm.at[pl.ds(base, CHA)], idxd)
        pltpu.async_copy(es_hbm.at[idxs], esv, sem).wait()
        pltpu.async_copy(ed_hbm.at[idxd], edv, sem).wait()

        def row(i, c2):
            t = esv[i, pl.ds(0, 16)] + edv[i, pl.ds(0, 16)]
            eev[i, pl.ds(0, 16)] = jnp.exp(jnp.maximum(t, 0.2 * t))
            for jj in range(1, 8):
                eev[i, pl.ds(jj * 16, 16)] = jnp.zeros((16,), jnp.float32)
            return c2

        lax.fori_loop(0, CHA, row, 0)
        pltpu.sync_copy(eev, ee_hbm.at[pl.ds(base, CHA), :])
        pltpu.sync_copy(eev, acc.at[idxd], add=True)
        return carry

    lax.fori_loop(0, NCHA, step, 0)
    plsc.subcore_barrier()
    _write_slab(den_hbm, zb, acc, cid, sid)


def _sc_att(es, ed, s_arr, d_arr, z128):
    k = pl.kernel(
        _sc_att_body,
        out_type=(
            jax.ShapeDtypeStruct((E, H), jnp.float32),
            jax.ShapeDtypeStruct((NCORE, NPAD, H), jnp.float32),
        ),
        mesh=_mesh(),
        scratch_types=[
            pltpu.VMEM((CHA,), jnp.int32),
            pltpu.VMEM((CHA,), jnp.int32),
            pltpu.VMEM((CHA, H), jnp.float32),
            pltpu.VMEM((CHA, H), jnp.float32),
            pltpu.VMEM((CHA, H), jnp.float32),
            pltpu.VMEM((80, H), jnp.float32),
            pltpu.VMEM_SHARED((NPAD, H), jnp.float32),
            pltpu.SemaphoreType.DMA,
        ],
    )
    return k(es, ed, s_arr, d_arr, z128)


def _sc_gat_body(hha_hbm, hhb_hbm, ee_hbm, inv_hbm, s_hbm, d_hbm, z_hbm,
                 out_hbm, idxs, idxd, hhv, eev, invv, mv, zb, acc, sem):
    cid = lax.axis_index("c")
    sid = lax.axis_index("s")
    wid = cid * NSUB + sid
    _zero_slab(z_hbm, zb, acc, sid)
    plsc.subcore_barrier()

    def step(k, carry):
        base = wid * EW + k * CHA
        pltpu.sync_copy(s_hbm.at[pl.ds(base, CHA)], idxs)
        pltpu.sync_copy(d_hbm.at[pl.ds(base, CHA)], idxd)
        pltpu.sync_copy(ee_hbm.at[pl.ds(base, CHA), :], eev)
        pltpu.async_copy(inv_hbm.at[idxd], invv, sem).wait()
        pltpu.async_copy(hha_hbm.at[idxs], hhv, sem).wait()

        def row_a(i, c2):
            att = eev[i, pl.ds(0, 16)] * invv[i, pl.ds(0, 16)]
            a0 = att[0]
            a1 = att[1]
            for jj in range(8):
                v = a0 * hhv[i, pl.ds(jj * 16, 16)]
                v = v + a1 * hhv[i, pl.ds(128 + jj * 16, 16)]
                mv[i, pl.ds(jj * 16, 16)] = v
            return c2

        lax.fori_loop(0, CHA, row_a, 0)
        pltpu.async_copy(hhb_hbm.at[idxs], hhv, sem).wait()

        def row_b(i, c2):
            att = eev[i, pl.ds(0, 16)] * invv[i, pl.ds(0, 16)]
            a2 = att[2]
            a3 = att[3]
            for jj in range(8):
                v = mv[i, pl.ds(jj * 16, 16)]
                v = v + a2 * hhv[i, pl.ds(jj * 16, 16)]
                v = v + a3 * hhv[i, pl.ds(128 + jj * 16, 16)]
                mv[i, pl.ds(jj * 16, 16)] = v
            return c2

        lax.fori_loop(0, CHA, row_b, 0)
        pltpu.sync_copy(mv, acc.at[idxd], add=True)
        return carry

    lax.fori_loop(0, NCHA, step, 0)
    plsc.subcore_barrier()
    _write_slab(out_hbm, zb, acc, cid, sid)


def _sc_gat(hha, hhb, ee, inv, s_arr, d_arr, z128):
    k = pl.kernel(
        _sc_gat_body,
        out_type=jax.ShapeDtypeStruct((NCORE, NPAD, H), jnp.float32),
        mesh=_mesh(),
        scratch_types=[
            pltpu.VMEM((CHA,), jnp.int32),
            pltpu.VMEM((CHA,), jnp.int32),
            pltpu.VMEM((CHA, 2 * H), jnp.float32),
            pltpu.VMEM((CHA, H), jnp.float32),
            pltpu.VMEM((CHA, H), jnp.float32),
            pltpu.VMEM((CHA, H), jnp.float32),
            pltpu.VMEM((80, H), jnp.float32),
            pltpu.VMEM_SHARED((NPAD, H), jnp.float32),
            pltpu.SemaphoreType.DMA,
        ],
    )
    return k(hha, hhb, ee, inv, s_arr, d_arr, z128)


def _dinv_of(d0, d1):
    return lax.rsqrt(1.0 + d0[:, 0:1] + d1[:, 0:1])


def _k1_body(x_ref, w_ref, d0_ref, d1_ref, o_ref):
    dinv = _dinv_of(d0_ref[...], d1_ref[...])
    o_ref[...] = jnp.dot(x_ref[...], w_ref[...],
                         preferred_element_type=jnp.float32) * dinv


def _k1(x, W, d0, d1):
    return pl.pallas_call(
        _k1_body,
        grid=(N // ROWS,),
        in_specs=[
            pl.BlockSpec((ROWS, D), lambda i: (i, 0)),
            pl.BlockSpec((D, H), lambda i: (0, 0)),
            pl.BlockSpec((ROWS, H), lambda i: (i, 0)),
            pl.BlockSpec((ROWS, H), lambda i: (i, 0)),
        ],
        out_specs=pl.BlockSpec((ROWS, H), lambda i: (i, 0)),
        out_shape=jax.ShapeDtypeStruct((N, H), jnp.float32),
    )(x, W, d0, d1)


def _k2_body(p0_ref, p1_ref, a_ref, d0_ref, d1_ref, b_ref, g_ref, be_ref,
             w_ref, o_ref):
    dinv = _dinv_of(d0_ref[...], d1_ref[...])
    hcol = (p0_ref[...] + p1_ref[...] + a_ref[...]) * dinv + b_ref[0:1, :]
    z = jnp.maximum(hcol * BNC * g_ref[0:1, :] + be_ref[0:1, :], 0.0)
    o_ref[...] = jnp.dot(z, w_ref[...],
                         preferred_element_type=jnp.float32) * dinv


def _k2(p0, p1, a, d0, d1, bpad, gpad, bepad, W):
    return pl.pallas_call(
        _k2_body,
        grid=(N // ROWS,),
        in_specs=[
            pl.BlockSpec((ROWS, H), lambda i: (i, 0)),
            pl.BlockSpec((ROWS, H), lambda i: (i, 0)),
            pl.BlockSpec((ROWS, H), lambda i: (i, 0)),
            pl.BlockSpec((ROWS, H), lambda i: (i, 0)),
            pl.BlockSpec((ROWS, H), lambda i: (i, 0)),
            pl.BlockSpec((8, H), lambda i: (0, 0)),
            pl.BlockSpec((8, H), lambda i: (0, 0)),
            pl.BlockSpec((8, H), lambda i: (0, 0)),
            pl.BlockSpec((H, H), lambda i: (0, 0)),
        ],
        out_specs=pl.BlockSpec((ROWS, H), lambda i: (i, 0)),
        out_shape=jax.ShapeDtypeStruct((N, H), jnp.float32),
    )(p0, p1, a, d0, d1, bpad, gpad, bepad, W)


def _k4_body(p0_ref, p1_ref, a_ref, d0_ref, d1_ref, b_ref, g_ref, be_ref,
             gwa_ref, gwb_ref, as_ref, ad_ref,
             hha_ref, hhb_ref, es_ref, ed_ref):
    dinv = _dinv_of(d0_ref[...], d1_ref[...])
    hcol = (p0_ref[...] + p1_ref[...] + a_ref[...]) * dinv + b_ref[0:1, :]
    z = jnp.maximum(hcol * BNC * g_ref[0:1, :] + be_ref[0:1, :], 0.0)
    hha = jnp.dot(z, gwa_ref[...], preferred_element_type=jnp.float32)
    hhb = jnp.dot(z, gwb_ref[...], preferred_element_type=jnp.float32)
    hha_ref[...] = hha
    hhb_ref[...] = hhb
    hh = jnp.concatenate([hha, hhb], axis=1)
    es_ref[...] = jnp.dot(hh, as_ref[...], preferred_element_type=jnp.float32)
    ed_ref[...] = jnp.dot(hh, ad_ref[...], preferred_element_type=jnp.float32)


def _k4(p0, p1, a, d0, d1, bpad, gpad, bepad, gatWa, gatWb, As, Ad):
    return pl.pallas_call(
        _k4_body,
        grid=(N // ROWS,),
        in_specs=[
            pl.BlockSpec((ROWS, H), lambda i: (i, 0)),
            pl.BlockSpec((ROWS, H), lambda i: (i, 0)),
            pl.BlockSpec((ROWS, H), lambda i: (i, 0)),
            pl.BlockSpec((ROWS, H), lambda i: (i, 0)),
            pl.BlockSpec((ROWS, H), lambda i: (i, 0)),
            pl.BlockSpec((8, H), lambda i: (0, 0)),
            pl.BlockSpec((8, H), lambda i: (0, 0)),
            pl.BlockSpec((8, H), lambda i: (0, 0)),
            pl.BlockSpec((H, 2 * H), lambda i: (0, 0)),
            pl.BlockSpec((H, 2 * H), lambda i: (0, 0)),
            pl.BlockSpec((4 * H, H), lambda i: (0, 0)),
            pl.BlockSpec((4 * H, H), lambda i: (0, 0)),
        ],
        out_specs=[
            pl.BlockSpec((ROWS, 2 * H), lambda i: (i, 0)),
            pl.BlockSpec((ROWS, 2 * H), lambda i: (i, 0)),
            pl.BlockSpec((ROWS, H), lambda i: (i, 0)),
            pl.BlockSpec((ROWS, H), lambda i: (i, 0)),
        ],
        out_shape=[
            jax.ShapeDtypeStruct((N, 2 * H), jnp.float32),
            jax.ShapeDtypeStruct((N, 2 * H), jnp.float32),
            jax.ShapeDtypeStruct((N, H), jnp.float32),
            jax.ShapeDtypeStruct((N, H), jnp.float32),
        ],
    )(p0, p1, a, d0, d1, bpad, gpad, bepad, gatWa, gatWb, As, Ad)


def _k5_body(dn0_ref, dn1_ref, es_ref, ed_ref, hha_ref, hhb_ref,
             inv_ref, base_ref):
    t = es_ref[...] + ed_ref[...]
    eself = jnp.exp(jnp.maximum(t, 0.2 * t))
    den = dn0_ref[...] + dn1_ref[...] + eself
    inv = 1.0 / (den + 1e-16)
    inv_ref[...] = inv
    w = eself * inv
    hha = hha_ref[...]
    hhb = hhb_ref[...]
    acc = w[:, 0:1] * hha[:, 0:H]
    acc = acc + w[:, 1:2] * hha[:, H:2 * H]
    acc = acc + w[:, 2:3] * hhb[:, 0:H]
    acc = acc + w[:, 3:4] * hhb[:, H:2 * H]
    base_ref[...] = acc


def _k5(dn0, dn1, es, ed, hha, hhb):
    return pl.pallas_call(
        _k5_body,
        grid=(N // ROWS,),
        in_specs=[
            pl.BlockSpec((ROWS, H), lambda i: (i, 0)),
            pl.BlockSpec((ROWS, H), lambda i: (i, 0)),
            pl.BlockSpec((ROWS, H), lambda i: (i, 0)),
            pl.BlockSpec((ROWS, H), lambda i: (i, 0)),
            pl.BlockSpec((ROWS, 2 * H), lambda i: (i, 0)),
            pl.BlockSpec((ROWS, 2 * H), lambda i: (i, 0)),
        ],
        out_specs=[
            pl.BlockSpec((ROWS, H), lambda i: (i, 0)),
            pl.BlockSpec((ROWS, H), lambda i: (i, 0)),
        ],
        out_shape=[
            jax.ShapeDtypeStruct((N, H), jnp.float32),
            jax.ShapeDtypeStruct((N, H), jnp.float32),
        ],
    )(dn0, dn1, es, ed, hha, hhb)


def _k6a_body(a0_ref, a1_ref, base_ref, gb_ref, brow_ref, bcol_ref,
              h4_ref, gcat_ref):
    h4 = jnp.maximum(
        (a0_ref[...] + a1_ref[...] + base_ref[...]) * 0.25 + gb_ref[0:1, :],
        0.0)
    h4_ref[...] = h4
    brow = brow_ref[0:1, :]
    gid = lax.broadcasted_iota(jnp.int32, (G, N), 0)
    oh = (jnp.broadcast_to(brow, (G, N)) == gid).astype(jnp.float32)
    gsum = jnp.dot(oh, h4, preferred_element_type=jnp.float32)
    cnt = jnp.sum(oh, axis=1, keepdims=True)
    gcat_ref[:, 0:H] = gsum / jnp.maximum(cnt, 1.0)
    bcol = bcol_ref[:, 0:1]
    nid = lax.broadcasted_iota(jnp.int32, (N, G), 1)
    ohn = jnp.broadcast_to(bcol, (N, G)) == nid
    for g in range(G):
        mg = jnp.max(jnp.where(ohn[:, g:g + 1], h4, -1e30), axis=0,
                     keepdims=True)
        gcat_ref[g:g + 1, H:2 * H] = mg


def _k6a(agg0, agg1, base, gbpad, brow8, bcol8):
    return pl.pallas_call(
        _k6a_body,
        grid=(1,),
        in_specs=[
            pl.BlockSpec((N, H), lambda i: (0, 0)),
            pl.BlockSpec((N, H), lambda i: (0, 0)),
            pl.BlockSpec((N, H), lambda i: (0, 0)),
            pl.BlockSpec((8, H), lambda i: (0, 0)),
            pl.BlockSpec((8, N), lambda i: (0, 0)),
            pl.BlockSpec((N, 8), lambda i: (0, 0)),
        ],
        out_specs=[
            pl.BlockSpec((N, H), lambda i: (0, 0)),
            pl.BlockSpec((G, 2 * H), lambda i: (0, 0)),
        ],
        out_shape=[
            jax.ShapeDtypeStruct((N, H), jnp.float32),
            jax.ShapeDtypeStruct((G, 2 * H), jnp.float32),
        ],
    )(agg0, agg1, base, gbpad, brow8, bcol8)


def _k6b_body(h4_ref, bcol_ref, gcat_ref, w1_ref, b1_ref,
              w2s_ref, w2l_ref, w2bp_ref, w2ad_ref,
              b2s_ref, b2l_ref, b2bp_ref, b2ad_ref,
              o0_ref, o1_ref, o2_ref, o3_ref):
    bcol = bcol_ref[:, 0:1]
    nid = lax.broadcasted_iota(jnp.int32, (ROWS, G), 1)
    ohb = (jnp.broadcast_to(bcol, (ROWS, G)) == nid).astype(jnp.float32)
    gf = jnp.dot(ohb, gcat_ref[...], preferred_element_type=jnp.float32)
    c = jnp.concatenate([h4_ref[...], gf], axis=1)
    z = jnp.maximum(jnp.dot(c, w1_ref[...],
                            preferred_element_type=jnp.float32)
                    + b1_ref[0:1, :], 0.0)
    o0_ref[...] = jnp.dot(z[:, 0:H], w2s_ref[...],
                          preferred_element_type=jnp.float32) + b2s_ref[0:1, :]
    o1_ref[...] = jnp.dot(z[:, H:2 * H], w2l_ref[...],
                          preferred_element_type=jnp.float32) + b2l_ref[0:1, :]
    obp = jnp.dot(z[:, 2 * H:3 * H], w2bp_ref[...],
                  preferred_element_type=jnp.float32) + b2bp_ref[0:1, :]
    o2_ref[...] = 1.0 / (1.0 + jnp.exp(-obp))
    oad = jnp.dot(z[:, 3 * H:4 * H], w2ad_ref[...],
                  preferred_element_type=jnp.float32) + b2ad_ref[0:1, :]
    colid = lax.broadcasted_iota(jnp.int32, (ROWS, H), 1)
    masked = jnp.where(colid < 4, oad, -1e30)
    mx = jnp.max(masked, axis=1, keepdims=True)
    ex = jnp.exp(masked - mx)
    o3_ref[...] = ex / jnp.sum(ex, axis=1, keepdims=True)


def _k6b(h4, bcol8, gcat, w1cat, b1cat, w2s, w2l, w2bp, w2ad,
         b2s, b2l, b2bp, b2ad):
    full = lambda r, c: pl.BlockSpec((r, c), lambda i: (0, 0))
    return pl.pallas_call(
        _k6b_body,
        grid=(N // ROWS,),
        in_specs=[
            pl.BlockSpec((ROWS, H), lambda i: (i, 0)),
            pl.BlockSpec((ROWS, 8), lambda i: (i, 0)),
            full(G, 2 * H),
            full(3 * H, 4 * H),
            full(8, 4 * H),
            full(H, H), full(H, H), full(H, H), full(H, H),
            full(8, H), full(8, H), full(8, H), full(8, H),
        ],
        out_specs=[pl.BlockSpec((ROWS, H), lambda i: (i, 0))
                   for _ in range(4)],
        out_shape=[jax.ShapeDtypeStruct((N, H), jnp.float32)
                   for _ in range(4)],
    )(h4, bcol8, gcat, w1cat, b1cat, w2s, w2l, w2bp, w2ad,
      b2s, b2l, b2bp, b2ad)


def _pad8(v):
    return jnp.broadcast_to(v[None, :], (8, v.shape[0]))


def kernel(x, edge_index, batch, W0, b0, g0, be0, W1, b1, g1, be1, W2, b2, g2, be2, gatW, asrc, adst, gatb, sW1, sb1, sW2, sb2, lW1, lb1, lW2, lb2, bpW1, bpb1, bpW2, bpb2, adW1, adb1, adW2, adb2):
    s_arr = edge_index[0]
    d_arr = edge_index[1]
    z128 = jnp.zeros((80, H), jnp.float32)
    ones128 = jnp.ones((CH, H), jnp.float32)

    degp = _sc_deg(d_arr, ones128, z128)
    d0 = degp[0, :N]
    d1 = degp[1, :N]

    a = _k1(x, W0, d0, d1)
    consts = [(b0, g0, be0), (b1, g1, be1)]
    for (bb, gg, bbe), W in zip(consts, (W1, W2)):
        p = _sc_gcn(a, s_arr, d_arr, z128)
        a = _k2(p[0, :N], p[1, :N], a, d0, d1,
                _pad8(bb), _pad8(gg), _pad8(bbe), W)

    p = _sc_gcn(a, s_arr, d_arr, z128)
    eye = jnp.eye(HEADS, H, dtype=jnp.float32)
    As = (asrc[:, :, None] * eye[:, None, :]).reshape(HEADS * H, H)
    Ad = (adst[:, :, None] * eye[:, None, :]).reshape(HEADS * H, H)
    hha, hhb, es, ed = _k4(p[0, :N], p[1, :N], a, d0, d1,
                           _pad8(b2), _pad8(g2), _pad8(be2),
                           gatW[:, :2 * H], gatW[:, 2 * H:], As, Ad)

    ee, denp = _sc_att(es, ed, s_arr, d_arr, z128)
    inv, base = _k5(denp[0, :N], denp[1, :N], es, ed, hha, hhb)
    aggp = _sc_gat(hha, hhb, ee, inv, s_arr, d_arr, z128)

    brow8 = jnp.broadcast_to(batch[None, :], (8, N))
    bcol8 = jnp.broadcast_to(batch[:, None], (N, 8))
    h4, gcat = _k6a(aggp[0, :N], aggp[1, :N], base, _pad8(gatb), brow8, bcol8)

    w1cat = jnp.concatenate([sW1, lW1, bpW1, adW1], axis=1)
    b1cat = _pad8(jnp.concatenate([sb1, lb1, bpb1, adb1]))
    padw = lambda w: jnp.zeros((H, H), jnp.float32).at[:, :w.shape[1]].set(w)
    padb = lambda b: jnp.zeros((H,), jnp.float32).at[:b.shape[0]].set(b)
    o0, o1, o2, o3 = _k6b(
        h4, bcol8, gcat, w1cat, b1cat,
        padw(sW2), padw(lW2), padw(bpW2), padw(adW2),
        _pad8(padb(sb2)), _pad8(padb(lb2)), _pad8(padb(bpb2)),
        _pad8(padb(adb2)))

    scales = o0[:, :2]
    layouts = o1[:, :4]
    breakpoints = o2[:, :1]
    adaptations = o3[:, :4]
    return scales, layouts, breakpoints, adaptations


# speedup vs baseline: 13.6770x; 1.1742x over previous
import math

import jax
import jax.numpy as jnp
from jax import lax
from jax.experimental import pallas as pl
from jax.experimental.pallas import tpu as pltpu
from jax.experimental.pallas import tpu_sc as plsc

N = 10000
E = 320000
D = 128
H = 128
HEADS = 4
G = 64

NCORE = 2
NSUB = 16
NW = NCORE * NSUB
EW = E // NW
CH = 80
NCH = EW // CH
CHA = 40
NCHA = EW // CHA
NPAD = 10240
SLAB = NPAD // NSUB
ROWS = 1000
BNC = 1.0 / math.sqrt(1.0 + 1e-5)


def _mesh():
    return plsc.VectorSubcoreMesh(
        core_axis_name="c", subcore_axis_name="s",
        num_cores=NCORE, num_subcores=NSUB)


def _zero_slab(z_hbm, zb, acc, sid):
    for q in range(SLAB // 80):
        pltpu.sync_copy(z_hbm, zb)
        pltpu.sync_copy(zb, acc.at[pl.ds(sid * SLAB + q * 80, 80), :])


def _write_slab(out_hbm, zb, acc, cid, sid):
    for q in range(SLAB // 80):
        r0 = sid * SLAB + q * 80
        pltpu.sync_copy(acc.at[pl.ds(r0, 80), :], zb)
        pltpu.sync_copy(zb, out_hbm.at[cid, pl.ds(r0, 80), :])


def _sc_deg_body(d_hbm, ones_hbm, z_hbm, out_hbm, idxd, ones_v, zb, acc, sem):
    cid = lax.axis_index("c")
    sid = lax.axis_index("s")
    wid = cid * NSUB + sid
    _zero_slab(z_hbm, zb, acc, sid)
    pltpu.sync_copy(ones_hbm, ones_v)
    plsc.subcore_barrier()

    def step(k, carry):
        base = wid * EW + k * CH
        pltpu.sync_copy(d_hbm.at[pl.ds(base, CH)], idxd)
        pltpu.sync_copy(ones_v, acc.at[idxd], add=True)
        return carry

    lax.fori_loop(0, NCH, step, 0)
    plsc.subcore_barrier()
    _write_slab(out_hbm, zb, acc, cid, sid)


def _sc_deg(d_arr, ones128, z128):
    k = pl.kernel(
        _sc_deg_body,
        out_type=jax.ShapeDtypeStruct((NCORE, NPAD, H), jnp.float32),
        mesh=_mesh(),
        scratch_types=[
            pltpu.VMEM((CH,), jnp.int32),
            pltpu.VMEM((CH, H), jnp.float32),
            pltpu.VMEM((80, H), jnp.float32),
            pltpu.VMEM_SHARED((NPAD, H), jnp.float32),
            pltpu.SemaphoreType.DMA,
        ],
    )
    return k(d_arr, ones128, z128)


def _sc_gcn_body(a_hbm, s_hbm, d_hbm, z_hbm, out_hbm,
                 idxs, idxd, rows, zb, acc, sem):
    cid = lax.axis_index("c")
    sid = lax.axis_index("s")
    wid = cid * NSUB + sid
    _zero_slab(z_hbm, zb, acc, sid)
    plsc.subcore_barrier()

    def step(k, carry):
        base = wid * EW + k * CH
        pltpu.sync_copy(s_hbm.at[pl.ds(base, CH)], idxs)
        pltpu.sync_copy(d_hbm.at[pl.ds(base, CH)], idxd)
        pltpu.async_copy(a_hbm.at[idxs], rows, sem).wait()
        pltpu.sync_copy(rows, acc.at[idxd], add=True)
        return carry

    lax.fori_loop(0, NCH, step, 0)
    plsc.subcore_barrier()
    _write_slab(out_hbm, zb, acc, cid, sid)


def _sc_gcn(a, s_arr, d_arr, z128):
    k = pl.kernel(
        _sc_gcn_body,
        out_type=jax.ShapeDtypeStruct((NCORE, NPAD, H), jnp.float32),
        mesh=_mesh(),
        scratch_types=[
            pltpu.VMEM((CH,), jnp.int32),
            pltpu.VMEM((CH,), jnp.int32),
            pltpu.VMEM((CH, H), jnp.float32),
            pltpu.VMEM((80, H), jnp.float32),
            pltpu.VMEM_SHARED((NPAD, H), jnp.float32),
            pltpu.SemaphoreType.DMA,
        ],
    )
    return k(a, s_arr, d_arr, z128)


def _sc_att_body(es_hbm, ed_hbm, s_hbm, d_hbm, z_hbm, ee_hbm, den_hbm,
                 idxs, idxd, esv, edv, eev, zb, acc, sem, sem2):
    cid = lax.axis_index("c")
    sid = lax.axis_index("s")
    wid = cid * NSUB + sid
    _zero_slab(z_hbm, zb, acc, sid)
    plsc.subcore_barrier()

    def step(k, carry):
        base = wid * EW + k * CHA
        pltpu.sync_copy(s_hbm.at[pl.ds(base, CHA)], idxs)
        pltpu.sync_copy(d_hbm.at[pl.ds(base, CHA)], idxd)
        cpa = pltpu.async_copy(es_hbm.at[idxs], esv, sem)
        cpb = pltpu.async_copy(ed_hbm.at[idxd], edv, sem2)
        cpa.wait()
        cpb.wait()

        def row(i, c2):
            t = esv[i, pl.ds(0, 16)] + edv[i, pl.ds(0, 16)]
            eev[i, pl.ds(0, 16)] = jnp.exp(jnp.maximum(t, 0.2 * t))
            for jj in range(1, 8):
                eev[i, pl.ds(jj * 16, 16)] = jnp.zeros((16,), jnp.float32)
            return c2

        lax.fori_loop(0, CHA, row, 0)
        pltpu.sync_copy(eev, ee_hbm.at[pl.ds(base, CHA), :])
        pltpu.sync_copy(eev, acc.at[idxd], add=True)
        return carry

    lax.fori_loop(0, NCHA, step, 0)
    plsc.subcore_barrier()
    _write_slab(den_hbm, zb, acc, cid, sid)


def _sc_att(es, ed, s_arr, d_arr, z128):
    k = pl.kernel(
        _sc_att_body,
        out_type=(
            jax.ShapeDtypeStruct((E, H), jnp.float32),
            jax.ShapeDtypeStruct((NCORE, NPAD, H), jnp.float32),
        ),
        mesh=_mesh(),
        scratch_types=[
            pltpu.VMEM((CHA,), jnp.int32),
            pltpu.VMEM((CHA,), jnp.int32),
            pltpu.VMEM((CHA, H), jnp.float32),
            pltpu.VMEM((CHA, H), jnp.float32),
            pltpu.VMEM((CHA, H), jnp.float32),
            pltpu.VMEM((80, H), jnp.float32),
            pltpu.VMEM_SHARED((NPAD, H), jnp.float32),
            pltpu.SemaphoreType.DMA,
            pltpu.SemaphoreType.DMA,
        ],
    )
    return k(es, ed, s_arr, d_arr, z128)


def _sc_gat_body(hha_hbm, hhb_hbm, ee_hbm, inv_hbm, s_hbm, d_hbm, z_hbm,
                 out_hbm, idxs, idxd, hhva, hhvb, eev, invv, mv, zb, acc,
                 sema, semb, semc):
    cid = lax.axis_index("c")
    sid = lax.axis_index("s")
    wid = cid * NSUB + sid
    _zero_slab(z_hbm, zb, acc, sid)
    plsc.subcore_barrier()

    def step(k, carry):
        base = wid * EW + k * CHA
        pltpu.sync_copy(s_hbm.at[pl.ds(base, CHA)], idxs)
        pltpu.sync_copy(d_hbm.at[pl.ds(base, CHA)], idxd)
        cpa = pltpu.async_copy(hha_hbm.at[idxs], hhva, sema)
        cpb = pltpu.async_copy(hhb_hbm.at[idxs], hhvb, semb)
        cpc = pltpu.async_copy(inv_hbm.at[idxd], invv, semc)
        pltpu.sync_copy(ee_hbm.at[pl.ds(base, CHA), :], eev)
        cpc.wait()
        cpa.wait()

        def row_a(i, c2):
            att = eev[i, pl.ds(0, 16)] * invv[i, pl.ds(0, 16)]
            a0 = att[0]
            a1 = att[1]
            for jj in range(8):
                v = a0 * hhva[i, pl.ds(jj * 16, 16)]
                v = v + a1 * hhva[i, pl.ds(128 + jj * 16, 16)]
                mv[i, pl.ds(jj * 16, 16)] = v
            return c2

        lax.fori_loop(0, CHA, row_a, 0)
        cpb.wait()

        def row_b(i, c2):
            att = eev[i, pl.ds(0, 16)] * invv[i, pl.ds(0, 16)]
            a2 = att[2]
            a3 = att[3]
            for jj in range(8):
                v = mv[i, pl.ds(jj * 16, 16)]
                v = v + a2 * hhvb[i, pl.ds(jj * 16, 16)]
                v = v + a3 * hhvb[i, pl.ds(128 + jj * 16, 16)]
                mv[i, pl.ds(jj * 16, 16)] = v
            return c2

        lax.fori_loop(0, CHA, row_b, 0)
        pltpu.sync_copy(mv, acc.at[idxd], add=True)
        return carry

    lax.fori_loop(0, NCHA, step, 0)
    plsc.subcore_barrier()
    _write_slab(out_hbm, zb, acc, cid, sid)


def _sc_gat(hha, hhb, ee, inv, s_arr, d_arr, z128):
    k = pl.kernel(
        _sc_gat_body,
        out_type=jax.ShapeDtypeStruct((NCORE, NPAD, H), jnp.float32),
        mesh=_mesh(),
        scratch_types=[
            pltpu.VMEM((CHA,), jnp.int32),
            pltpu.VMEM((CHA,), jnp.int32),
            pltpu.VMEM((CHA, 2 * H), jnp.float32),
            pltpu.VMEM((CHA, 2 * H), jnp.float32),
            pltpu.VMEM((CHA, H), jnp.float32),
            pltpu.VMEM((CHA, H), jnp.float32),
            pltpu.VMEM((CHA, H), jnp.float32),
            pltpu.VMEM((80, H), jnp.float32),
            pltpu.VMEM_SHARED((NPAD, H), jnp.float32),
            pltpu.SemaphoreType.DMA,
            pltpu.SemaphoreType.DMA,
            pltpu.SemaphoreType.DMA,
        ],
    )
    return k(hha, hhb, ee, inv, s_arr, d_arr, z128)


def _dinv_of(d0, d1):
    return lax.rsqrt(1.0 + d0[:, 0:1] + d1[:, 0:1])


def _k1_body(x_ref, w_ref, d0_ref, d1_ref, o_ref):
    dinv = _dinv_of(d0_ref[...], d1_ref[...])
    o_ref[...] = jnp.dot(x_ref[...], w_ref[...],
                         preferred_element_type=jnp.float32) * dinv


def _k1(x, W, d0, d1):
    return pl.pallas_call(
        _k1_body,
        grid=(N // ROWS,),
        in_specs=[
            pl.BlockSpec((ROWS, D), lambda i: (i, 0)),
            pl.BlockSpec((D, H), lambda i: (0, 0)),
            pl.BlockSpec((ROWS, H), lambda i: (i, 0)),
            pl.BlockSpec((ROWS, H), lambda i: (i, 0)),
        ],
        out_specs=pl.BlockSpec((ROWS, H), lambda i: (i, 0)),
        out_shape=jax.ShapeDtypeStruct((N, H), jnp.float32),
    )(x, W, d0, d1)


def _k2_body(p0_ref, p1_ref, a_ref, d0_ref, d1_ref, b_ref, g_ref, be_ref,
             w_ref, o_ref):
    dinv = _dinv_of(d0_ref[...], d1_ref[...])
    hcol = (p0_ref[...] + p1_ref[...] + a_ref[...]) * dinv + b_ref[0:1, :]
    z = jnp.maximum(hcol * BNC * g_ref[0:1, :] + be_ref[0:1, :], 0.0)
    o_ref[...] = jnp.dot(z, w_ref[...],
                         preferred_element_type=jnp.float32) * dinv


def _k2(p0, p1, a, d0, d1, bpad, gpad, bepad, W):
    return pl.pallas_call(
        _k2_body,
        grid=(N // ROWS,),
        in_specs=[
            pl.BlockSpec((ROWS, H), lambda i: (i, 0)),
            pl.BlockSpec((ROWS, H), lambda i: (i, 0)),
            pl.BlockSpec((ROWS, H), lambda i: (i, 0)),
            pl.BlockSpec((ROWS, H), lambda i: (i, 0)),
            pl.BlockSpec((ROWS, H), lambda i: (i, 0)),
            pl.BlockSpec((8, H), lambda i: (0, 0)),
            pl.BlockSpec((8, H), lambda i: (0, 0)),
            pl.BlockSpec((8, H), lambda i: (0, 0)),
            pl.BlockSpec((H, H), lambda i: (0, 0)),
        ],
        out_specs=pl.BlockSpec((ROWS, H), lambda i: (i, 0)),
        out_shape=jax.ShapeDtypeStruct((N, H), jnp.float32),
    )(p0, p1, a, d0, d1, bpad, gpad, bepad, W)


def _k4_body(p0_ref, p1_ref, a_ref, d0_ref, d1_ref, b_ref, g_ref, be_ref,
             gwa_ref, gwb_ref, as_ref, ad_ref,
             hha_ref, hhb_ref, es_ref, ed_ref):
    dinv = _dinv_of(d0_ref[...], d1_ref[...])
    hcol = (p0_ref[...] + p1_ref[...] + a_ref[...]) * dinv + b_ref[0:1, :]
    z = jnp.maximum(hcol * BNC * g_ref[0:1, :] + be_ref[0:1, :], 0.0)
    hha = jnp.dot(z, gwa_ref[...], preferred_element_type=jnp.float32)
    hhb = jnp.dot(z, gwb_ref[...], preferred_element_type=jnp.float32)
    hha_ref[...] = hha
    hhb_ref[...] = hhb
    hh = jnp.concatenate([hha, hhb], axis=1)
    es_ref[...] = jnp.dot(hh, as_ref[...], preferred_element_type=jnp.float32)
    ed_ref[...] = jnp.dot(hh, ad_ref[...], preferred_element_type=jnp.float32)


def _k4(p0, p1, a, d0, d1, bpad, gpad, bepad, gatWa, gatWb, As, Ad):
    return pl.pallas_call(
        _k4_body,
        grid=(N // ROWS,),
        in_specs=[
            pl.BlockSpec((ROWS, H), lambda i: (i, 0)),
            pl.BlockSpec((ROWS, H), lambda i: (i, 0)),
            pl.BlockSpec((ROWS, H), lambda i: (i, 0)),
            pl.BlockSpec((ROWS, H), lambda i: (i, 0)),
            pl.BlockSpec((ROWS, H), lambda i: (i, 0)),
            pl.BlockSpec((8, H), lambda i: (0, 0)),
            pl.BlockSpec((8, H), lambda i: (0, 0)),
            pl.BlockSpec((8, H), lambda i: (0, 0)),
            pl.BlockSpec((H, 2 * H), lambda i: (0, 0)),
            pl.BlockSpec((H, 2 * H), lambda i: (0, 0)),
            pl.BlockSpec((4 * H, H), lambda i: (0, 0)),
            pl.BlockSpec((4 * H, H), lambda i: (0, 0)),
        ],
        out_specs=[
            pl.BlockSpec((ROWS, 2 * H), lambda i: (i, 0)),
            pl.BlockSpec((ROWS, 2 * H), lambda i: (i, 0)),
            pl.BlockSpec((ROWS, H), lambda i: (i, 0)),
            pl.BlockSpec((ROWS, H), lambda i: (i, 0)),
        ],
        out_shape=[
            jax.ShapeDtypeStruct((N, 2 * H), jnp.float32),
            jax.ShapeDtypeStruct((N, 2 * H), jnp.float32),
            jax.ShapeDtypeStruct((N, H), jnp.float32),
            jax.ShapeDtypeStruct((N, H), jnp.float32),
        ],
    )(p0, p1, a, d0, d1, bpad, gpad, bepad, gatWa, gatWb, As, Ad)


def _k5_body(dn0_ref, dn1_ref, es_ref, ed_ref, hha_ref, hhb_ref,
             inv_ref, base_ref):
    t = es_ref[...] + ed_ref[...]
    eself = jnp.exp(jnp.maximum(t, 0.2 * t))
    den = dn0_ref[...] + dn1_ref[...] + eself
    inv = 1.0 / (den + 1e-16)
    inv_ref[...] = inv
    w = eself * inv
    hha = hha_ref[...]
    hhb = hhb_ref[...]
    acc = w[:, 0:1] * hha[:, 0:H]
    acc = acc + w[:, 1:2] * hha[:, H:2 * H]
    acc = acc + w[:, 2:3] * hhb[:, 0:H]
    acc = acc + w[:, 3:4] * hhb[:, H:2 * H]
    base_ref[...] = acc


def _k5(dn0, dn1, es, ed, hha, hhb):
    return pl.pallas_call(
        _k5_body,
        grid=(N // ROWS,),
        in_specs=[
            pl.BlockSpec((ROWS, H), lambda i: (i, 0)),
            pl.BlockSpec((ROWS, H), lambda i: (i, 0)),
            pl.BlockSpec((ROWS, H), lambda i: (i, 0)),
            pl.BlockSpec((ROWS, H), lambda i: (i, 0)),
            pl.BlockSpec((ROWS, 2 * H), lambda i: (i, 0)),
            pl.BlockSpec((ROWS, 2 * H), lambda i: (i, 0)),
        ],
        out_specs=[
            pl.BlockSpec((ROWS, H), lambda i: (i, 0)),
            pl.BlockSpec((ROWS, H), lambda i: (i, 0)),
        ],
        out_shape=[
            jax.ShapeDtypeStruct((N, H), jnp.float32),
            jax.ShapeDtypeStruct((N, H), jnp.float32),
        ],
    )(dn0, dn1, es, ed, hha, hhb)


def _k6a_body(a0_ref, a1_ref, base_ref, gb_ref, brow_ref, bcol_ref,
              h4_ref, gcat_ref):
    h4 = jnp.maximum(
        (a0_ref[...] + a1_ref[...] + base_ref[...]) * 0.25 + gb_ref[0:1, :],
        0.0)
    h4_ref[...] = h4
    brow = brow_ref[0:1, :]
    gid = lax.broadcasted_iota(jnp.int32, (G, N), 0)
    oh = (jnp.broadcast_to(brow, (G, N)) == gid).astype(jnp.float32)
    gsum = jnp.dot(oh, h4, preferred_element_type=jnp.float32)
    cnt = jnp.sum(oh, axis=1, keepdims=True)
    gcat_ref[:, 0:H] = gsum / jnp.maximum(cnt, 1.0)
    bcol = bcol_ref[:, 0:1]
    nid = lax.broadcasted_iota(jnp.int32, (N, G), 1)
    ohn = jnp.broadcast_to(bcol, (N, G)) == nid
    for g in range(G):
        mg = jnp.max(jnp.where(ohn[:, g:g + 1], h4, -1e30), axis=0,
                     keepdims=True)
        gcat_ref[g:g + 1, H:2 * H] = mg


def _k6a(agg0, agg1, base, gbpad, brow8, bcol8):
    return pl.pallas_call(
        _k6a_body,
        grid=(1,),
        in_specs=[
            pl.BlockSpec((N, H), lambda i: (0, 0)),
            pl.BlockSpec((N, H), lambda i: (0, 0)),
            pl.BlockSpec((N, H), lambda i: (0, 0)),
            pl.BlockSpec((8, H), lambda i: (0, 0)),
            pl.BlockSpec((8, N), lambda i: (0, 0)),
            pl.BlockSpec((N, 8), lambda i: (0, 0)),
        ],
        out_specs=[
            pl.BlockSpec((N, H), lambda i: (0, 0)),
            pl.BlockSpec((G, 2 * H), lambda i: (0, 0)),
        ],
        out_shape=[
            jax.ShapeDtypeStruct((N, H), jnp.float32),
            jax.ShapeDtypeStruct((G, 2 * H), jnp.float32),
        ],
    )(agg0, agg1, base, gbpad, brow8, bcol8)


def _k6b_body(h4_ref, bcol_ref, gcat_ref, w1_ref, b1_ref,
              w2s_ref, w2l_ref, w2bp_ref, w2ad_ref,
              b2s_ref, b2l_ref, b2bp_ref, b2ad_ref,
              o0_ref, o1_ref, o2_ref, o3_ref):
    bcol = bcol_ref[:, 0:1]
    nid = lax.broadcasted_iota(jnp.int32, (ROWS, G), 1)
    ohb = (jnp.broadcast_to(bcol, (ROWS, G)) == nid).astype(jnp.float32)
    gf = jnp.dot(ohb, gcat_ref[...], preferred_element_type=jnp.float32)
    c = jnp.concatenate([h4_ref[...], gf], axis=1)
    z = jnp.maximum(jnp.dot(c, w1_ref[...],
                            preferred_element_type=jnp.float32)
                    + b1_ref[0:1, :], 0.0)
    o0_ref[...] = jnp.dot(z[:, 0:H], w2s_ref[...],
                          preferred_element_type=jnp.float32) + b2s_ref[0:1, :]
    o1_ref[...] = jnp.dot(z[:, H:2 * H], w2l_ref[...],
                          preferred_element_type=jnp.float32) + b2l_ref[0:1, :]
    obp = jnp.dot(z[:, 2 * H:3 * H], w2bp_ref[...],
                  preferred_element_type=jnp.float32) + b2bp_ref[0:1, :]
    o2_ref[...] = 1.0 / (1.0 + jnp.exp(-obp))
    oad = jnp.dot(z[:, 3 * H:4 * H], w2ad_ref[...],
                  preferred_element_type=jnp.float32) + b2ad_ref[0:1, :]
    colid = lax.broadcasted_iota(jnp.int32, (ROWS, H), 1)
    masked = jnp.where(colid < 4, oad, -1e30)
    mx = jnp.max(masked, axis=1, keepdims=True)
    ex = jnp.exp(masked - mx)
    o3_ref[...] = ex / jnp.sum(ex, axis=1, keepdims=True)


def _k6b(h4, bcol8, gcat, w1cat, b1cat, w2s, w2l, w2bp, w2ad,
         b2s, b2l, b2bp, b2ad):
    full = lambda r, c: pl.BlockSpec((r, c), lambda i: (0, 0))
    return pl.pallas_call(
        _k6b_body,
        grid=(N // ROWS,),
        in_specs=[
            pl.BlockSpec((ROWS, H), lambda i: (i, 0)),
            pl.BlockSpec((ROWS, 8), lambda i: (i, 0)),
            full(G, 2 * H),
            full(3 * H, 4 * H),
            full(8, 4 * H),
            full(H, H), full(H, H), full(H, H), full(H, H),
            full(8, H), full(8, H), full(8, H), full(8, H),
        ],
        out_specs=[pl.BlockSpec((ROWS, H), lambda i: (i, 0))
                   for _ in range(4)],
        out_shape=[jax.ShapeDtypeStruct((N, H), jnp.float32)
                   for _ in range(4)],
    )(h4, bcol8, gcat, w1cat, b1cat, w2s, w2l, w2bp, w2ad,
      b2s, b2l, b2bp, b2ad)


def _pad8(v):
    return jnp.broadcast_to(v[None, :], (8, v.shape[0]))


def kernel(x, edge_index, batch, W0, b0, g0, be0, W1, b1, g1, be1, W2, b2, g2, be2, gatW, asrc, adst, gatb, sW1, sb1, sW2, sb2, lW1, lb1, lW2, lb2, bpW1, bpb1, bpW2, bpb2, adW1, adb1, adW2, adb2):
    s_arr = edge_index[0]
    d_arr = edge_index[1]
    z128 = jnp.zeros((80, H), jnp.float32)
    ones128 = jnp.ones((CH, H), jnp.float32)

    degp = _sc_deg(d_arr, ones128, z128)
    d0 = degp[0, :N]
    d1 = degp[1, :N]

    a = _k1(x, W0, d0, d1)
    consts = [(b0, g0, be0), (b1, g1, be1)]
    for (bb, gg, bbe), W in zip(consts, (W1, W2)):
        p = _sc_gcn(a, s_arr, d_arr, z128)
        a = _k2(p[0, :N], p[1, :N], a, d0, d1,
                _pad8(bb), _pad8(gg), _pad8(bbe), W)

    p = _sc_gcn(a, s_arr, d_arr, z128)
    eye = jnp.eye(HEADS, H, dtype=jnp.float32)
    As = (asrc[:, :, None] * eye[:, None, :]).reshape(HEADS * H, H)
    Ad = (adst[:, :, None] * eye[:, None, :]).reshape(HEADS * H, H)
    hha, hhb, es, ed = _k4(p[0, :N], p[1, :N], a, d0, d1,
                           _pad8(b2), _pad8(g2), _pad8(be2),
                           gatW[:, :2 * H], gatW[:, 2 * H:], As, Ad)

    ee, denp = _sc_att(es, ed, s_arr, d_arr, z128)
    inv, base = _k5(denp[0, :N], denp[1, :N], es, ed, hha, hhb)
    aggp = _sc_gat(hha, hhb, ee, inv, s_arr, d_arr, z128)

    brow8 = jnp.broadcast_to(batch[None, :], (8, N))
    bcol8 = jnp.broadcast_to(batch[:, None], (N, 8))
    h4, gcat = _k6a(aggp[0, :N], aggp[1, :N], base, _pad8(gatb), brow8, bcol8)

    w1cat = jnp.concatenate([sW1, lW1, bpW1, adW1], axis=1)
    b1cat = _pad8(jnp.concatenate([sb1, lb1, bpb1, adb1]))
    padw = lambda w: jnp.zeros((H, H), jnp.float32).at[:, :w.shape[1]].set(w)
    padb = lambda b: jnp.zeros((H,), jnp.float32).at[:b.shape[0]].set(b)
    o0, o1, o2, o3 = _k6b(
        h4, bcol8, gcat, w1cat, b1cat,
        padw(sW2), padw(lW2), padw(bpW2), padw(adW2),
        _pad8(padb(sb2)), _pad8(padb(lb2)), _pad8(padb(bpb2)),
        _pad8(padb(adb2)))

    scales = o0[:, :2]
    layouts = o1[:, :4]
    breakpoints = o2[:, :1]
    adaptations = o3[:, :4]
    return scales, layouts, breakpoints, adaptations


# speedup vs baseline: 13.6995x; 1.0016x over previous
import math

import jax
import jax.numpy as jnp
from jax import lax
from jax.experimental import pallas as pl
from jax.experimental.pallas import tpu as pltpu
from jax.experimental.pallas import tpu_sc as plsc

N = 10000
E = 320000
D = 128
H = 128
HEADS = 4
G = 64

NCORE = 2
NSUB = 16
NW = NCORE * NSUB
EW = E // NW
CH = 80
NCH = EW // CH
CHA = 40
NCHA = EW // CHA
NPAD = 10240
SLAB = NPAD // NSUB
ROWS = 1000
BNC = 1.0 / math.sqrt(1.0 + 1e-5)


def _mesh():
    return plsc.VectorSubcoreMesh(
        core_axis_name="c", subcore_axis_name="s",
        num_cores=NCORE, num_subcores=NSUB)


def _zero_slab(z_hbm, zb, acc, sid):
    for q in range(SLAB // 80):
        pltpu.sync_copy(z_hbm, zb)
        pltpu.sync_copy(zb, acc.at[pl.ds(sid * SLAB + q * 80, 80), :])


def _write_slab(out_hbm, zb, acc, cid, sid):
    for q in range(SLAB // 80):
        r0 = sid * SLAB + q * 80
        pltpu.sync_copy(acc.at[pl.ds(r0, 80), :], zb)
        pltpu.sync_copy(zb, out_hbm.at[cid, pl.ds(r0, 80), :])


def _sc_deg_body(d_hbm, ones_hbm, z_hbm, out_hbm, idxd, ones_v, zb, acc, sem):
    cid = lax.axis_index("c")
    sid = lax.axis_index("s")
    wid = cid * NSUB + sid
    _zero_slab(z_hbm, zb, acc, sid)
    pltpu.sync_copy(ones_hbm, ones_v)
    plsc.subcore_barrier()

    def step(k, carry):
        base = wid * EW + k * CH
        pltpu.sync_copy(d_hbm.at[pl.ds(base, CH)], idxd)
        pltpu.sync_copy(ones_v, acc.at[idxd], add=True)
        return carry

    lax.fori_loop(0, NCH, step, 0)
    plsc.subcore_barrier()
    _write_slab(out_hbm, zb, acc, cid, sid)


def _sc_deg(d_arr, ones128, z128):
    k = pl.kernel(
        _sc_deg_body,
        out_type=jax.ShapeDtypeStruct((NCORE, NPAD, H), jnp.float32),
        mesh=_mesh(),
        scratch_types=[
            pltpu.VMEM((CH,), jnp.int32),
            pltpu.VMEM((CH, H), jnp.float32),
            pltpu.VMEM((80, H), jnp.float32),
            pltpu.VMEM_SHARED((NPAD, H), jnp.float32),
            pltpu.SemaphoreType.DMA,
        ],
    )
    return k(d_arr, ones128, z128)


def _sc_gcn_body(a_hbm, s_hbm, d_hbm, z_hbm, out_hbm,
                 idxs, idxd, rows, zb, acc, sem):
    cid = lax.axis_index("c")
    sid = lax.axis_index("s")
    wid = cid * NSUB + sid
    _zero_slab(z_hbm, zb, acc, sid)
    plsc.subcore_barrier()

    def step(k, carry):
        base = wid * EW + k * CH
        pltpu.sync_copy(s_hbm.at[pl.ds(base, CH)], idxs)
        pltpu.sync_copy(d_hbm.at[pl.ds(base, CH)], idxd)
        pltpu.async_copy(a_hbm.at[idxs], rows, sem).wait()
        pltpu.sync_copy(rows, acc.at[idxd], add=True)
        return carry

    lax.fori_loop(0, NCH, step, 0)
    plsc.subcore_barrier()
    _write_slab(out_hbm, zb, acc, cid, sid)


def _sc_gcn(a, s_arr, d_arr, z128):
    k = pl.kernel(
        _sc_gcn_body,
        out_type=jax.ShapeDtypeStruct((NCORE, NPAD, H), jnp.float32),
        mesh=_mesh(),
        scratch_types=[
            pltpu.VMEM((CH,), jnp.int32),
            pltpu.VMEM((CH,), jnp.int32),
            pltpu.VMEM((CH, H), jnp.float32),
            pltpu.VMEM((80, H), jnp.float32),
            pltpu.VMEM_SHARED((NPAD, H), jnp.float32),
            pltpu.SemaphoreType.DMA,
        ],
    )
    return k(a, s_arr, d_arr, z128)


def _sc_att_body(es_hbm, ed_hbm, s_hbm, d_hbm, z_hbm, ee_hbm, den_hbm,
                 idxs, idxd, esv, edv, eev, eep, zb, acc, sem, sem2):
    cid = lax.axis_index("c")
    sid = lax.axis_index("s")
    wid = cid * NSUB + sid
    _zero_slab(z_hbm, zb, acc, sid)
    plsc.subcore_barrier()

    def step(k, carry):
        base = wid * EW + k * CHA
        pltpu.sync_copy(s_hbm.at[pl.ds(base, CHA)], idxs)
        pltpu.sync_copy(d_hbm.at[pl.ds(base, CHA)], idxd)
        cpa = pltpu.async_copy(es_hbm.at[idxs], esv, sem)
        cpb = pltpu.async_copy(ed_hbm.at[idxd], edv, sem2)
        cpa.wait()
        cpb.wait()

        def row(i, c2):
            t = esv[i, pl.ds(0, 16)] + edv[i, pl.ds(0, 16)]
            e = jnp.exp(jnp.maximum(t, 0.2 * t))
            eev[i, pl.ds(0, 16)] = e
            for jj in range(1, 8):
                eev[i, pl.ds(jj * 16, 16)] = jnp.zeros((16,), jnp.float32)
            eep[i // 8, pl.ds((i % 8) * 16, 16)] = e
            return c2

        lax.fori_loop(0, CHA, row, 0)
        pltpu.sync_copy(eep, ee_hbm.at[pl.ds((wid * NCHA + k) * 8, 8), :])
        pltpu.sync_copy(eev, acc.at[idxd], add=True)
        return carry

    lax.fori_loop(0, NCHA, step, 0)
    plsc.subcore_barrier()
    _write_slab(den_hbm, zb, acc, cid, sid)


def _sc_att(es, ed, s_arr, d_arr, z128):
    k = pl.kernel(
        _sc_att_body,
        out_type=(
            jax.ShapeDtypeStruct((NW * NCHA * 8, H), jnp.float32),
            jax.ShapeDtypeStruct((NCORE, NPAD, H), jnp.float32),
        ),
        mesh=_mesh(),
        scratch_types=[
            pltpu.VMEM((CHA,), jnp.int32),
            pltpu.VMEM((CHA,), jnp.int32),
            pltpu.VMEM((CHA, H), jnp.float32),
            pltpu.VMEM((CHA, H), jnp.float32),
            pltpu.VMEM((CHA, H), jnp.float32),
            pltpu.VMEM((8, H), jnp.float32),
            pltpu.VMEM((80, H), jnp.float32),
            pltpu.VMEM_SHARED((NPAD, H), jnp.float32),
            pltpu.SemaphoreType.DMA,
            pltpu.SemaphoreType.DMA,
        ],
    )
    return k(es, ed, s_arr, d_arr, z128)


def _sc_gat_body(hha_hbm, hhb_hbm, ee_hbm, inv_hbm, s_hbm, d_hbm, z_hbm,
                 out_hbm, idxs, idxd, hhva, hhvb, eev, invv, mv, zb, acc,
                 sema, semb, semc):
    cid = lax.axis_index("c")
    sid = lax.axis_index("s")
    wid = cid * NSUB + sid
    _zero_slab(z_hbm, zb, acc, sid)
    plsc.subcore_barrier()

    def step(k, carry):
        base = wid * EW + k * CHA
        pltpu.sync_copy(s_hbm.at[pl.ds(base, CHA)], idxs)
        pltpu.sync_copy(d_hbm.at[pl.ds(base, CHA)], idxd)
        cpa = pltpu.async_copy(hha_hbm.at[idxs], hhva, sema)
        cpb = pltpu.async_copy(hhb_hbm.at[idxs], hhvb, semb)
        cpc = pltpu.async_copy(inv_hbm.at[idxd], invv, semc)
        pltpu.sync_copy(ee_hbm.at[pl.ds((wid * NCHA + k) * 8, 8), :], eev)
        cpc.wait()
        cpa.wait()

        def row_a(i, c2):
            att = eev[i // 8, pl.ds((i % 8) * 16, 16)] * invv[i, pl.ds(0, 16)]
            a0 = att[0]
            a1 = att[1]
            for jj in range(8):
                v = a0 * hhva[i, pl.ds(jj * 16, 16)]
                v = v + a1 * hhva[i, pl.ds(128 + jj * 16, 16)]
                mv[i, pl.ds(jj * 16, 16)] = v
            return c2

        lax.fori_loop(0, CHA, row_a, 0)
        cpb.wait()

        def row_b(i, c2):
            att = eev[i // 8, pl.ds((i % 8) * 16, 16)] * invv[i, pl.ds(0, 16)]
            a2 = att[2]
            a3 = att[3]
            for jj in range(8):
                v = mv[i, pl.ds(jj * 16, 16)]
                v = v + a2 * hhvb[i, pl.ds(jj * 16, 16)]
                v = v + a3 * hhvb[i, pl.ds(128 + jj * 16, 16)]
                mv[i, pl.ds(jj * 16, 16)] = v
            return c2

        lax.fori_loop(0, CHA, row_b, 0)
        pltpu.sync_copy(mv, acc.at[idxd], add=True)
        return carry

    lax.fori_loop(0, NCHA, step, 0)
    plsc.subcore_barrier()
    _write_slab(out_hbm, zb, acc, cid, sid)


def _sc_gat(hha, hhb, ee, inv, s_arr, d_arr, z128):
    k = pl.kernel(
        _sc_gat_body,
        out_type=jax.ShapeDtypeStruct((NCORE, NPAD, H), jnp.float32),
        mesh=_mesh(),
        scratch_types=[
            pltpu.VMEM((CHA,), jnp.int32),
            pltpu.VMEM((CHA,), jnp.int32),
            pltpu.VMEM((CHA, 2 * H), jnp.float32),
            pltpu.VMEM((CHA, 2 * H), jnp.float32),
            pltpu.VMEM((8, H), jnp.float32),
            pltpu.VMEM((CHA, H), jnp.float32),
            pltpu.VMEM((CHA, H), jnp.float32),
            pltpu.VMEM((80, H), jnp.float32),
            pltpu.VMEM_SHARED((NPAD, H), jnp.float32),
            pltpu.SemaphoreType.DMA,
            pltpu.SemaphoreType.DMA,
            pltpu.SemaphoreType.DMA,
        ],
    )
    return k(hha, hhb, ee, inv, s_arr, d_arr, z128)


def _dinv_of(d0, d1):
    return lax.rsqrt(1.0 + d0[:, 0:1] + d1[:, 0:1])


def _k1_body(x_ref, w_ref, d0_ref, d1_ref, o_ref):
    dinv = _dinv_of(d0_ref[...], d1_ref[...])
    o_ref[...] = jnp.dot(x_ref[...], w_ref[...],
                         preferred_element_type=jnp.float32) * dinv


def _k1(x, W, d0, d1):
    return pl.pallas_call(
        _k1_body,
        grid=(N // ROWS,),
        in_specs=[
            pl.BlockSpec((ROWS, D), lambda i: (i, 0)),
            pl.BlockSpec((D, H), lambda i: (0, 0)),
            pl.BlockSpec((ROWS, H), lambda i: (i, 0)),
            pl.BlockSpec((ROWS, H), lambda i: (i, 0)),
        ],
        out_specs=pl.BlockSpec((ROWS, H), lambda i: (i, 0)),
        out_shape=jax.ShapeDtypeStruct((N, H), jnp.float32),
    )(x, W, d0, d1)


def _k2_body(p0_ref, p1_ref, a_ref, d0_ref, d1_ref, b_ref, g_ref, be_ref,
             w_ref, o_ref):
    dinv = _dinv_of(d0_ref[...], d1_ref[...])
    hcol = (p0_ref[...] + p1_ref[...] + a_ref[...]) * dinv + b_ref[0:1, :]
    z = jnp.maximum(hcol * BNC * g_ref[0:1, :] + be_ref[0:1, :], 0.0)
    o_ref[...] = jnp.dot(z, w_ref[...],
                         preferred_element_type=jnp.float32) * dinv


def _k2(p0, p1, a, d0, d1, bpad, gpad, bepad, W):
    return pl.pallas_call(
        _k2_body,
        grid=(N // ROWS,),
        in_specs=[
            pl.BlockSpec((ROWS, H), lambda i: (i, 0)),
            pl.BlockSpec((ROWS, H), lambda i: (i, 0)),
            pl.BlockSpec((ROWS, H), lambda i: (i, 0)),
            pl.BlockSpec((ROWS, H), lambda i: (i, 0)),
            pl.BlockSpec((ROWS, H), lambda i: (i, 0)),
            pl.BlockSpec((8, H), lambda i: (0, 0)),
            pl.BlockSpec((8, H), lambda i: (0, 0)),
            pl.BlockSpec((8, H), lambda i: (0, 0)),
            pl.BlockSpec((H, H), lambda i: (0, 0)),
        ],
        out_specs=pl.BlockSpec((ROWS, H), lambda i: (i, 0)),
        out_shape=jax.ShapeDtypeStruct((N, H), jnp.float32),
    )(p0, p1, a, d0, d1, bpad, gpad, bepad, W)


def _k4_body(p0_ref, p1_ref, a_ref, d0_ref, d1_ref, b_ref, g_ref, be_ref,
             gwa_ref, gwb_ref, as_ref, ad_ref,
             hha_ref, hhb_ref, es_ref, ed_ref):
    dinv = _dinv_of(d0_ref[...], d1_ref[...])
    hcol = (p0_ref[...] + p1_ref[...] + a_ref[...]) * dinv + b_ref[0:1, :]
    z = jnp.maximum(hcol * BNC * g_ref[0:1, :] + be_ref[0:1, :], 0.0)
    hha = jnp.dot(z, gwa_ref[...], preferred_element_type=jnp.float32)
    hhb = jnp.dot(z, gwb_ref[...], preferred_element_type=jnp.float32)
    hha_ref[...] = hha
    hhb_ref[...] = hhb
    hh = jnp.concatenate([hha, hhb], axis=1)
    es_ref[...] = jnp.dot(hh, as_ref[...], preferred_element_type=jnp.float32)
    ed_ref[...] = jnp.dot(hh, ad_ref[...], preferred_element_type=jnp.float32)


def _k4(p0, p1, a, d0, d1, bpad, gpad, bepad, gatWa, gatWb, As, Ad):
    return pl.pallas_call(
        _k4_body,
        grid=(N // ROWS,),
        in_specs=[
            pl.BlockSpec((ROWS, H), lambda i: (i, 0)),
            pl.BlockSpec((ROWS, H), lambda i: (i, 0)),
            pl.BlockSpec((ROWS, H), lambda i: (i, 0)),
            pl.BlockSpec((ROWS, H), lambda i: (i, 0)),
            pl.BlockSpec((ROWS, H), lambda i: (i, 0)),
            pl.BlockSpec((8, H), lambda i: (0, 0)),
            pl.BlockSpec((8, H), lambda i: (0, 0)),
            pl.BlockSpec((8, H), lambda i: (0, 0)),
            pl.BlockSpec((H, 2 * H), lambda i: (0, 0)),
            pl.BlockSpec((H, 2 * H), lambda i: (0, 0)),
            pl.BlockSpec((4 * H, H), lambda i: (0, 0)),
            pl.BlockSpec((4 * H, H), lambda i: (0, 0)),
        ],
        out_specs=[
            pl.BlockSpec((ROWS, 2 * H), lambda i: (i, 0)),
            pl.BlockSpec((ROWS, 2 * H), lambda i: (i, 0)),
            pl.BlockSpec((ROWS, H), lambda i: (i, 0)),
            pl.BlockSpec((ROWS, H), lambda i: (i, 0)),
        ],
        out_shape=[
            jax.ShapeDtypeStruct((N, 2 * H), jnp.float32),
            jax.ShapeDtypeStruct((N, 2 * H), jnp.float32),
            jax.ShapeDtypeStruct((N, H), jnp.float32),
            jax.ShapeDtypeStruct((N, H), jnp.float32),
        ],
    )(p0, p1, a, d0, d1, bpad, gpad, bepad, gatWa, gatWb, As, Ad)


def _k5_body(dn0_ref, dn1_ref, es_ref, ed_ref, hha_ref, hhb_ref,
             inv_ref, base_ref):
    t = es_ref[...] + ed_ref[...]
    eself = jnp.exp(jnp.maximum(t, 0.2 * t))
    den = dn0_ref[...] + dn1_ref[...] + eself
    inv = 1.0 / (den + 1e-16)
    inv_ref[...] = inv
    w = eself * inv
    hha = hha_ref[...]
    hhb = hhb_ref[...]
    acc = w[:, 0:1] * hha[:, 0:H]
    acc = acc + w[:, 1:2] * hha[:, H:2 * H]
    acc = acc + w[:, 2:3] * hhb[:, 0:H]
    acc = acc + w[:, 3:4] * hhb[:, H:2 * H]
    base_ref[...] = acc


def _k5(dn0, dn1, es, ed, hha, hhb):
    return pl.pallas_call(
        _k5_body,
        grid=(N // ROWS,),
        in_specs=[
            pl.BlockSpec((ROWS, H), lambda i: (i, 0)),
            pl.BlockSpec((ROWS, H), lambda i: (i, 0)),
            pl.BlockSpec((ROWS, H), lambda i: (i, 0)),
            pl.BlockSpec((ROWS, H), lambda i: (i, 0)),
            pl.BlockSpec((ROWS, 2 * H), lambda i: (i, 0)),
            pl.BlockSpec((ROWS, 2 * H), lambda i: (i, 0)),
        ],
        out_specs=[
            pl.BlockSpec((ROWS, H), lambda i: (i, 0)),
            pl.BlockSpec((ROWS, H), lambda i: (i, 0)),
        ],
        out_shape=[
            jax.ShapeDtypeStruct((N, H), jnp.float32),
            jax.ShapeDtypeStruct((N, H), jnp.float32),
        ],
    )(dn0, dn1, es, ed, hha, hhb)


def _k6a_body(a0_ref, a1_ref, base_ref, gb_ref, brow_ref, bcol_ref,
              h4_ref, gcat_ref):
    h4 = jnp.maximum(
        (a0_ref[...] + a1_ref[...] + base_ref[...]) * 0.25 + gb_ref[0:1, :],
        0.0)
    h4_ref[...] = h4
    brow = brow_ref[0:1, :]
    gid = lax.broadcasted_iota(jnp.int32, (G, N), 0)
    oh = (jnp.broadcast_to(brow, (G, N)) == gid).astype(jnp.float32)
    gsum = jnp.dot(oh, h4, preferred_element_type=jnp.float32)
    cnt = jnp.sum(oh, axis=1, keepdims=True)
    gcat_ref[:, 0:H] = gsum / jnp.maximum(cnt, 1.0)
    bcol = bcol_ref[:, 0:1]
    nid = lax.broadcasted_iota(jnp.int32, (N, G), 1)
    ohn = jnp.broadcast_to(bcol, (N, G)) == nid
    for g in range(G):
        mg = jnp.max(jnp.where(ohn[:, g:g + 1], h4, -1e30), axis=0,
                     keepdims=True)
        gcat_ref[g:g + 1, H:2 * H] = mg


def _k6a(agg0, agg1, base, gbpad, brow8, bcol8):
    return pl.pallas_call(
        _k6a_body,
        grid=(1,),
        in_specs=[
            pl.BlockSpec((N, H), lambda i: (0, 0)),
            pl.BlockSpec((N, H), lambda i: (0, 0)),
            pl.BlockSpec((N, H), lambda i: (0, 0)),
            pl.BlockSpec((8, H), lambda i: (0, 0)),
            pl.BlockSpec((8, N), lambda i: (0, 0)),
            pl.BlockSpec((N, 8), lambda i: (0, 0)),
        ],
        out_specs=[
            pl.BlockSpec((N, H), lambda i: (0, 0)),
            pl.BlockSpec((G, 2 * H), lambda i: (0, 0)),
        ],
        out_shape=[
            jax.ShapeDtypeStruct((N, H), jnp.float32),
            jax.ShapeDtypeStruct((G, 2 * H), jnp.float32),
        ],
    )(agg0, agg1, base, gbpad, brow8, bcol8)


def _k6b_body(h4_ref, bcol_ref, gcat_ref, w1_ref, b1_ref,
              w2s_ref, w2l_ref, w2bp_ref, w2ad_ref,
              b2s_ref, b2l_ref, b2bp_ref, b2ad_ref,
              o0_ref, o1_ref, o2_ref, o3_ref):
    bcol = bcol_ref[:, 0:1]
    nid = lax.broadcasted_iota(jnp.int32, (ROWS, G), 1)
    ohb = (jnp.broadcast_to(bcol, (ROWS, G)) == nid).astype(jnp.float32)
    gf = jnp.dot(ohb, gcat_ref[...], preferred_element_type=jnp.float32)
    c = jnp.concatenate([h4_ref[...], gf], axis=1)
    z = jnp.maximum(jnp.dot(c, w1_ref[...],
                            preferred_element_type=jnp.float32)
                    + b1_ref[0:1, :], 0.0)
    o0_ref[...] = jnp.dot(z[:, 0:H], w2s_ref[...],
                          preferred_element_type=jnp.float32) + b2s_ref[0:1, :]
    o1_ref[...] = jnp.dot(z[:, H:2 * H], w2l_ref[...],
                          preferred_element_type=jnp.float32) + b2l_ref[0:1, :]
    obp = jnp.dot(z[:, 2 * H:3 * H], w2bp_ref[...],
                  preferred_element_type=jnp.float32) + b2bp_ref[0:1, :]
    o2_ref[...] = 1.0 / (1.0 + jnp.exp(-obp))
    oad = jnp.dot(z[:, 3 * H:4 * H], w2ad_ref[...],
                  preferred_element_type=jnp.float32) + b2ad_ref[0:1, :]
    colid = lax.broadcasted_iota(jnp.int32, (ROWS, H), 1)
    masked = jnp.where(colid < 4, oad, -1e30)
    mx = jnp.max(masked, axis=1, keepdims=True)
    ex = jnp.exp(masked - mx)
    o3_ref[...] = ex / jnp.sum(ex, axis=1, keepdims=True)


def _k6b(h4, bcol8, gcat, w1cat, b1cat, w2s, w2l, w2bp, w2ad,
         b2s, b2l, b2bp, b2ad):
    full = lambda r, c: pl.BlockSpec((r, c), lambda i: (0, 0))
    return pl.pallas_call(
        _k6b_body,
        grid=(N // ROWS,),
        in_specs=[
            pl.BlockSpec((ROWS, H), lambda i: (i, 0)),
            pl.BlockSpec((ROWS, 8), lambda i: (i, 0)),
            full(G, 2 * H),
            full(3 * H, 4 * H),
            full(8, 4 * H),
            full(H, H), full(H, H), full(H, H), full(H, H),
            full(8, H), full(8, H), full(8, H), full(8, H),
        ],
        out_specs=[pl.BlockSpec((ROWS, H), lambda i: (i, 0))
                   for _ in range(4)],
        out_shape=[jax.ShapeDtypeStruct((N, H), jnp.float32)
                   for _ in range(4)],
    )(h4, bcol8, gcat, w1cat, b1cat, w2s, w2l, w2bp, w2ad,
      b2s, b2l, b2bp, b2ad)


def _pad8(v):
    return jnp.broadcast_to(v[None, :], (8, v.shape[0]))


def kernel(x, edge_index, batch, W0, b0, g0, be0, W1, b1, g1, be1, W2, b2, g2, be2, gatW, asrc, adst, gatb, sW1, sb1, sW2, sb2, lW1, lb1, lW2, lb2, bpW1, bpb1, bpW2, bpb2, adW1, adb1, adW2, adb2):
    s_arr = edge_index[0]
    d_arr = edge_index[1]
    z128 = jnp.zeros((80, H), jnp.float32)
    ones128 = jnp.ones((CH, H), jnp.float32)

    degp = _sc_deg(d_arr, ones128, z128)
    d0 = degp[0, :N]
    d1 = degp[1, :N]

    a = _k1(x, W0, d0, d1)
    consts = [(b0, g0, be0), (b1, g1, be1)]
    for (bb, gg, bbe), W in zip(consts, (W1, W2)):
        p = _sc_gcn(a, s_arr, d_arr, z128)
        a = _k2(p[0, :N], p[1, :N], a, d0, d1,
                _pad8(bb), _pad8(gg), _pad8(bbe), W)

    p = _sc_gcn(a, s_arr, d_arr, z128)
    eye = jnp.eye(HEADS, H, dtype=jnp.float32)
    As = (asrc[:, :, None] * eye[:, None, :]).reshape(HEADS * H, H)
    Ad = (adst[:, :, None] * eye[:, None, :]).reshape(HEADS * H, H)
    hha, hhb, es, ed = _k4(p[0, :N], p[1, :N], a, d0, d1,
                           _pad8(b2), _pad8(g2), _pad8(be2),
                           gatW[:, :2 * H], gatW[:, 2 * H:], As, Ad)

    ee, denp = _sc_att(es, ed, s_arr, d_arr, z128)
    inv, base = _k5(denp[0, :N], denp[1, :N], es, ed, hha, hhb)
    aggp = _sc_gat(hha, hhb, ee, inv, s_arr, d_arr, z128)

    brow8 = jnp.broadcast_to(batch[None, :], (8, N))
    bcol8 = jnp.broadcast_to(batch[:, None], (N, 8))
    h4, gcat = _k6a(aggp[0, :N], aggp[1, :N], base, _pad8(gatb), brow8, bcol8)

    w1cat = jnp.concatenate([sW1, lW1, bpW1, adW1], axis=1)
    b1cat = _pad8(jnp.concatenate([sb1, lb1, bpb1, adb1]))
    padw = lambda w: jnp.zeros((H, H), jnp.float32).at[:, :w.shape[1]].set(w)
    padb = lambda b: jnp.zeros((H,), jnp.float32).at[:b.shape[0]].set(b)
    o0, o1, o2, o3 = _k6b(
        h4, bcol8, gcat, w1cat, b1cat,
        padw(sW2), padw(lW2), padw(bpW2), padw(adW2),
        _pad8(padb(sb2)), _pad8(padb(lb2)), _pad8(padb(bpb2)),
        _pad8(padb(adb2)))

    scales = o0[:, :2]
    layouts = o1[:, :4]
    breakpoints = o2[:, :1]
    adaptations = o3[:, :4]
    return scales, layouts, breakpoints, adaptations
